# Initial kernel scaffold; baseline (speedup 1.0000x reference)
import jax
import jax.numpy as jnp
from jax.experimental import pallas as pl


def kernel(x, edge_index, Wl1, bl1, Wr1, Wl2, bl2, Wr2):
    raise NotImplementedError("write your pallas kernel here")


# speedup vs baseline: 6.1585x; 6.1585x over previous
import functools

import jax
import jax.numpy as jnp
from jax import lax
from jax.experimental import pallas as pl
from jax.experimental.pallas import tpu as pltpu
from jax.experimental.pallas import tpu_sc as plsc

N = 10000
E = 320000
IN_D = 128
HID = 64
OUT_D = 32

NC = 2
NS = 16
NW = NC * NS
BATCH = 128
NB = 80
EDGES_PER_W = NB * BATCH
E_PAD = NW * EDGES_PER_W
N_PAD = 10240
ROWS_PER_TILE = N_PAD // NS
RB = 128
DUMMY = N

BM = 1000
GRID = N // BM

_mesh = plsc.VectorSubcoreMesh(core_axis_name="c", subcore_axis_name="s")


def _edge_pass(table, src3, dst3, width, with_deg):
    out_types = [jax.ShapeDtypeStruct((NC, N_PAD, width), jnp.float32)]
    scratch = [
        pltpu.VMEM((NB, BATCH), jnp.int32),
        pltpu.VMEM((NB, BATCH), jnp.int32),
        pltpu.VMEM((BATCH, width), jnp.float32),
        pltpu.VMEM((RB, width), jnp.float32),
        pltpu.VMEM_SHARED((N_PAD, width), jnp.float32),
    ]
    if with_deg:
        out_types.append(jax.ShapeDtypeStruct((NC, N_PAD, 16), jnp.float32))
        scratch += [
            pltpu.VMEM((BATCH, 16), jnp.float32),
            pltpu.VMEM((RB, 16), jnp.float32),
            pltpu.VMEM_SHARED((N_PAD, 16), jnp.float32),
        ]

    def body(table_hbm, src_hbm, dst_hbm, *refs):
        if with_deg:
            (agg_hbm, deg_hbm, srcv, dstv, rows, zb, acc, ones, dzb, dacc) = refs
        else:
            (agg_hbm, srcv, dstv, rows, zb, acc) = refs
        c = lax.axis_index("c")
        s = lax.axis_index("s")
        w = c * NS + s
        base = s * ROWS_PER_TILE

        @pl.loop(0, RB)
        def _(i):
            for k in range(width // 16):
                zb[i, pl.ds(k * 16, 16)] = jnp.zeros((16,), jnp.float32)
            if with_deg:
                dzb[i, pl.ds(0, 16)] = jnp.zeros((16,), jnp.float32)
                ones[i, pl.ds(0, 16)] = jnp.ones((16,), jnp.float32)

        @pl.loop(0, ROWS_PER_TILE, step=RB)
        def _(r):
            pltpu.sync_copy(zb, acc.at[pl.ds(base + r, RB)])
            if with_deg:
                pltpu.sync_copy(dzb, dacc.at[pl.ds(base + r, RB)])

        pltpu.sync_copy(src_hbm.at[w], srcv)
        pltpu.sync_copy(dst_hbm.at[w], dstv)
        plsc.subcore_barrier()

        @pl.loop(0, NB)
        def _(j):
            pltpu.sync_copy(table_hbm.at[srcv.at[j]], rows)
            pltpu.sync_copy(rows, acc.at[dstv.at[j]], add=True)
            if with_deg:
                pltpu.sync_copy(ones, dacc.at[dstv.at[j]], add=True)

        plsc.subcore_barrier()

        pltpu.sync_copy(acc.at[pl.ds(base, ROWS_PER_TILE)],
                        agg_hbm.at[c, pl.ds(base, ROWS_PER_TILE)])
        if with_deg:
            pltpu.sync_copy(dacc.at[pl.ds(base, ROWS_PER_TILE)],
                            deg_hbm.at[c, pl.ds(base, ROWS_PER_TILE)])

    kern = pl.kernel(body, out_type=out_types, mesh=_mesh,
                     scratch_types=scratch,
                     compiler_params=pltpu.CompilerParams(
                         use_tc_tiling_on_sc=False))
    return kern(table, src3, dst3)


def _proj1_body(x_ref, w_ref, p_ref, r_ref):
    y = jnp.dot(x_ref[...], w_ref[...], preferred_element_type=jnp.float32)
    p_ref[...] = y[:, :HID]
    r_ref[...] = y[:, HID:]


def _combine1_body(a_ref, d_ref, r1_ref, w2_ref, b1_ref,
                   p2_ref, r2_ref, inv_ref):
    agg = a_ref[0] + a_ref[1]
    deg = d_ref[0] + d_ref[1]
    inv = 1.0 / jnp.maximum(deg, 1.0)
    h = jnp.maximum(agg * inv[:, 0:1] + r1_ref[...] + b1_ref[...], 0.0)
    pr = jnp.dot(h, w2_ref[...], preferred_element_type=jnp.float32)
    p2_ref[...] = pr[:, :OUT_D]
    r2_ref[...] = pr[:, OUT_D:]
    inv_ref[...] = inv


def _combine2_body(a_ref, inv_ref, r2_ref, b2_ref, o_ref):
    agg = a_ref[0] + a_ref[1]
    o_ref[...] = agg * inv_ref[:, 0:1] + r2_ref[...] + b2_ref[...]


def kernel(x, edge_index, Wl1, bl1, Wr1, Wl2, bl2, Wr2):
    src = edge_index[0].astype(jnp.int32)
    dst = edge_index[1].astype(jnp.int32)
    pad = E_PAD - E
    src3 = jnp.concatenate([src, jnp.zeros((pad,), jnp.int32)]).reshape(
        NW, NB, BATCH)
    dst3 = jnp.concatenate([dst, jnp.full((pad,), DUMMY, jnp.int32)]).reshape(
        NW, NB, BATCH)

    w1 = jnp.concatenate([Wl1, Wr1], axis=1)
    w2 = jnp.concatenate([Wl2, Wr2], axis=1)
    b1 = bl1.reshape(1, HID)
    b2 = bl2.reshape(1, OUT_D)

    p1, r1 = pl.pallas_call(
        _proj1_body,
        grid=(GRID,),
        in_specs=[
            pl.BlockSpec((BM, IN_D), lambda i: (i, 0)),
            pl.BlockSpec((IN_D, IN_D), lambda i: (0, 0)),
        ],
        out_specs=[
            pl.BlockSpec((BM, HID), lambda i: (i, 0)),
            pl.BlockSpec((BM, HID), lambda i: (i, 0)),
        ],
        out_shape=[
            jax.ShapeDtypeStruct((N, HID), jnp.float32),
            jax.ShapeDtypeStruct((N, HID), jnp.float32),
        ],
    )(x, w1)

    agg1p, degp = _edge_pass(p1, src3, dst3, HID, True)

    p2, r2, invd = pl.pallas_call(
        _combine1_body,
        grid=(GRID,),
        in_specs=[
            pl.BlockSpec((NC, BM, HID), lambda i: (0, i, 0)),
            pl.BlockSpec((NC, BM, 16), lambda i: (0, i, 0)),
            pl.BlockSpec((BM, HID), lambda i: (i, 0)),
            pl.BlockSpec((HID, HID), lambda i: (0, 0)),
            pl.BlockSpec((1, HID), lambda i: (0, 0)),
        ],
        out_specs=[
            pl.BlockSpec((BM, OUT_D), lambda i: (i, 0)),
            pl.BlockSpec((BM, OUT_D), lambda i: (i, 0)),
            pl.BlockSpec((BM, 16), lambda i: (i, 0)),
        ],
        out_shape=[
            jax.ShapeDtypeStruct((N, OUT_D), jnp.float32),
            jax.ShapeDtypeStruct((N, OUT_D), jnp.float32),
            jax.ShapeDtypeStruct((N, 16), jnp.float32),
        ],
    )(agg1p, degp, r1, w2, b1)

    (agg2p,) = _edge_pass(p2, src3, dst3, OUT_D, False)

    out = pl.pallas_call(
        _combine2_body,
        grid=(GRID,),
        in_specs=[
            pl.BlockSpec((NC, BM, OUT_D), lambda i: (0, i, 0)),
            pl.BlockSpec((BM, 16), lambda i: (i, 0)),
            pl.BlockSpec((BM, OUT_D), lambda i: (i, 0)),
            pl.BlockSpec((1, OUT_D), lambda i: (0, 0)),
        ],
        out_specs=pl.BlockSpec((BM, OUT_D), lambda i: (i, 0)),
        out_shape=jax.ShapeDtypeStruct((N, OUT_D), jnp.float32),
    )(agg2p, invd, r2, b2)

    return out


# speedup vs baseline: 7.4431x; 1.2086x over previous
import functools

import jax
import jax.numpy as jnp
from jax import lax
from jax.experimental import pallas as pl
from jax.experimental.pallas import tpu as pltpu
from jax.experimental.pallas import tpu_sc as plsc

N = 10000
E = 320000
IN_D = 128
HID = 64
OUT_D = 32

NC = 2
NS = 16
NW = NC * NS
BATCH = 128
NB = 80
EDGES_PER_W = NB * BATCH
E_PAD = NW * EDGES_PER_W
N_PAD = 10240
ROWS_PER_TILE = N_PAD // NS
RB = 128
NBUF = 4
DUMMY = N

BM = 1000
GRID = N // BM

_mesh = plsc.VectorSubcoreMesh(core_axis_name="c", subcore_axis_name="s")


def _edge_pass(table, src3, dst3, width, with_deg):
    out_types = [jax.ShapeDtypeStruct((NC, N_PAD, width), jnp.float32)]
    scratch = [
        pltpu.VMEM((NB, BATCH), jnp.int32),
        pltpu.VMEM((NB, BATCH), jnp.int32),
        pltpu.VMEM((NBUF, BATCH, width), jnp.float32),
        pltpu.VMEM((RB, width), jnp.float32),
        pltpu.VMEM_SHARED((N_PAD, width), jnp.float32),
        pltpu.SemaphoreType.DMA((NBUF,)),
        pltpu.SemaphoreType.DMA((NBUF,)),
        pltpu.SemaphoreType.DMA,
    ]
    if with_deg:
        out_types.append(jax.ShapeDtypeStruct((NC, N_PAD, 16), jnp.float32))
        scratch += [
            pltpu.VMEM((BATCH, 16), jnp.float32),
            pltpu.VMEM((RB, 16), jnp.float32),
            pltpu.VMEM_SHARED((N_PAD, 16), jnp.float32),
        ]

    def body(table_hbm, src_hbm, dst_hbm, *refs):
        if with_deg:
            (agg_hbm, deg_hbm, srcv, dstv, rows, zb, acc,
             g_sem, s_sem, o_sem, ones, dzb, dacc) = refs
        else:
            (agg_hbm, srcv, dstv, rows, zb, acc, g_sem, s_sem, o_sem) = refs
        c = lax.axis_index("c")
        s = lax.axis_index("s")
        w = c * NS + s
        base = s * ROWS_PER_TILE

        @pl.loop(0, RB)
        def _(i):
            for k in range(width // 16):
                zb[i, pl.ds(k * 16, 16)] = jnp.zeros((16,), jnp.float32)
            if with_deg:
                dzb[i, pl.ds(0, 16)] = jnp.zeros((16,), jnp.float32)
                ones[i, pl.ds(0, 16)] = jnp.ones((16,), jnp.float32)

        @pl.loop(0, ROWS_PER_TILE, step=RB)
        def _(r):
            pltpu.sync_copy(zb, acc.at[pl.ds(base + r, RB)])
            if with_deg:
                pltpu.sync_copy(dzb, dacc.at[pl.ds(base + r, RB)])

        pltpu.sync_copy(src_hbm.at[w], srcv)
        pltpu.sync_copy(dst_hbm.at[w], dstv)
        plsc.subcore_barrier()

        @pl.loop(0, NB, step=NBUF)
        def _(j0):
            gds = []
            for b in range(NBUF):
                @pl.when(j0 > 0)
                def _(b=b):
                    pltpu.make_async_copy(
                        rows.at[b], acc.at[dstv.at[j0]], s_sem.at[b]).wait()
                gds.append(pltpu.async_copy(
                    table_hbm.at[srcv.at[j0 + b]], rows.at[b], g_sem.at[b]))
            for b in range(NBUF):
                gds[b].wait()
                pltpu.async_copy(rows.at[b], acc.at[dstv.at[j0 + b]],
                                 s_sem.at[b], add=True)
                if with_deg:
                    od = pltpu.async_copy(ones, dacc.at[dstv.at[j0 + b]],
                                          o_sem, add=True)

                    @pl.when(j0 > 0)
                    def _(od=od):
                        od.wait()

        for b in range(NBUF):
            pltpu.make_async_copy(rows.at[b], acc.at[dstv.at[0]],
                                  s_sem.at[b]).wait()
            if with_deg:
                pltpu.make_async_copy(ones, dacc.at[dstv.at[0]], o_sem).wait()

        plsc.subcore_barrier()

        pltpu.sync_copy(acc.at[pl.ds(base, ROWS_PER_TILE)],
                        agg_hbm.at[c, pl.ds(base, ROWS_PER_TILE)])
        if with_deg:
            pltpu.sync_copy(dacc.at[pl.ds(base, ROWS_PER_TILE)],
                            deg_hbm.at[c, pl.ds(base, ROWS_PER_TILE)])

    kern = pl.kernel(body, out_type=out_types, mesh=_mesh,
                     scratch_types=scratch,
                     compiler_params=pltpu.CompilerParams(
                         use_tc_tiling_on_sc=False))
    return kern(table, src3, dst3)


def _proj1_body(x_ref, w_ref, p_ref, r_ref):
    y = jnp.dot(x_ref[...], w_ref[...], preferred_element_type=jnp.float32)
    p_ref[...] = y[:, :HID]
    r_ref[...] = y[:, HID:]


def _combine1_body(a_ref, d_ref, r1_ref, w2_ref, b1_ref,
                   p2_ref, r2_ref, inv_ref):
    agg = a_ref[0] + a_ref[1]
    deg = d_ref[0] + d_ref[1]
    inv = 1.0 / jnp.maximum(deg, 1.0)
    h = jnp.maximum(agg * inv[:, 0:1] + r1_ref[...] + b1_ref[...], 0.0)
    pr = jnp.dot(h, w2_ref[...], preferred_element_type=jnp.float32)
    p2_ref[...] = pr[:, :OUT_D]
    r2_ref[...] = pr[:, OUT_D:]
    inv_ref[...] = inv


def _combine2_body(a_ref, inv_ref, r2_ref, b2_ref, o_ref):
    agg = a_ref[0] + a_ref[1]
    o_ref[...] = agg * inv_ref[:, 0:1] + r2_ref[...] + b2_ref[...]


def kernel(x, edge_index, Wl1, bl1, Wr1, Wl2, bl2, Wr2):
    src = edge_index[0].astype(jnp.int32)
    dst = edge_index[1].astype(jnp.int32)
    pad = E_PAD - E
    src3 = jnp.concatenate([src, jnp.zeros((pad,), jnp.int32)]).reshape(
        NW, NB, BATCH)
    dst3 = jnp.concatenate([dst, jnp.full((pad,), DUMMY, jnp.int32)]).reshape(
        NW, NB, BATCH)

    w1 = jnp.concatenate([Wl1, Wr1], axis=1)
    w2 = jnp.concatenate([Wl2, Wr2], axis=1)
    b1 = bl1.reshape(1, HID)
    b2 = bl2.reshape(1, OUT_D)

    p1, r1 = pl.pallas_call(
        _proj1_body,
        grid=(GRID,),
        in_specs=[
            pl.BlockSpec((BM, IN_D), lambda i: (i, 0)),
            pl.BlockSpec((IN_D, IN_D), lambda i: (0, 0)),
        ],
        out_specs=[
            pl.BlockSpec((BM, HID), lambda i: (i, 0)),
            pl.BlockSpec((BM, HID), lambda i: (i, 0)),
        ],
        out_shape=[
            jax.ShapeDtypeStruct((N, HID), jnp.float32),
            jax.ShapeDtypeStruct((N, HID), jnp.float32),
        ],
    )(x, w1)

    agg1p, degp = _edge_pass(p1, src3, dst3, HID, True)

    p2, r2, invd = pl.pallas_call(
        _combine1_body,
        grid=(GRID,),
        in_specs=[
            pl.BlockSpec((NC, BM, HID), lambda i: (0, i, 0)),
            pl.BlockSpec((NC, BM, 16), lambda i: (0, i, 0)),
            pl.BlockSpec((BM, HID), lambda i: (i, 0)),
            pl.BlockSpec((HID, HID), lambda i: (0, 0)),
            pl.BlockSpec((1, HID), lambda i: (0, 0)),
        ],
        out_specs=[
            pl.BlockSpec((BM, OUT_D), lambda i: (i, 0)),
            pl.BlockSpec((BM, OUT_D), lambda i: (i, 0)),
            pl.BlockSpec((BM, 16), lambda i: (i, 0)),
        ],
        out_shape=[
            jax.ShapeDtypeStruct((N, OUT_D), jnp.float32),
            jax.ShapeDtypeStruct((N, OUT_D), jnp.float32),
            jax.ShapeDtypeStruct((N, 16), jnp.float32),
        ],
    )(agg1p, degp, r1, w2, b1)

    (agg2p,) = _edge_pass(p2, src3, dst3, OUT_D, False)

    out = pl.pallas_call(
        _combine2_body,
        grid=(GRID,),
        in_specs=[
            pl.BlockSpec((NC, BM, OUT_D), lambda i: (0, i, 0)),
            pl.BlockSpec((BM, 16), lambda i: (i, 0)),
            pl.BlockSpec((BM, OUT_D), lambda i: (i, 0)),
            pl.BlockSpec((1, OUT_D), lambda i: (0, 0)),
        ],
        out_specs=pl.BlockSpec((BM, OUT_D), lambda i: (i, 0)),
        out_shape=jax.ShapeDtypeStruct((N, OUT_D), jnp.float32),
    )(agg2p, invd, r2, b2)

    return out


# speedup vs baseline: 7.7250x; 1.0379x over previous
import functools

import jax
import jax.numpy as jnp
from jax import lax
from jax.experimental import pallas as pl
from jax.experimental.pallas import tpu as pltpu
from jax.experimental.pallas import tpu_sc as plsc

N = 10000
E = 320000
IN_D = 128
HID = 64
OUT_D = 32

NC = 2
NS = 16
BATCH = 128
B_FAST = 124
B_SLOW = 36
FAST_C = 0
NBT = 16 * B_FAST + 16 * B_SLOW
NBT_PAD = 16 * B_FAST + 15 * B_SLOW + B_FAST
N_PAD = 10240
ROWS_PER_TILE = N_PAD // NS
RB = 128
NBUF = 4
DUMMY = N

BM = 1000
GRID = N // BM

_mesh = plsc.VectorSubcoreMesh(core_axis_name="c", subcore_axis_name="s")


def _edge_pass(table, src3, dst3, width, with_deg):
    out_types = [jax.ShapeDtypeStruct((NC, N_PAD, width), jnp.float32)]
    scratch = [
        pltpu.VMEM((B_FAST, BATCH), jnp.int32),
        pltpu.VMEM((B_FAST, BATCH), jnp.int32),
        pltpu.VMEM((NBUF, BATCH, width), jnp.float32),
        pltpu.VMEM((RB, width), jnp.float32),
        pltpu.VMEM_SHARED((N_PAD, width), jnp.float32),
        pltpu.SemaphoreType.DMA((NBUF,)),
        pltpu.SemaphoreType.DMA((NBUF,)),
        pltpu.SemaphoreType.DMA,
    ]
    if with_deg:
        out_types.append(jax.ShapeDtypeStruct((NC, N_PAD, 16), jnp.float32))
        scratch += [
            pltpu.VMEM((BATCH, 16), jnp.float32),
            pltpu.VMEM((RB, 16), jnp.float32),
            pltpu.VMEM_SHARED((N_PAD, 16), jnp.float32),
        ]

    def body(table_hbm, src_hbm, dst_hbm, *refs):
        if with_deg:
            (agg_hbm, deg_hbm, srcv, dstv, rows, zb, acc,
             g_sem, s_sem, o_sem, ones, dzb, dacc) = refs
        else:
            (agg_hbm, srcv, dstv, rows, zb, acc, g_sem, s_sem, o_sem) = refs
        c = lax.axis_index("c")
        s = lax.axis_index("s")
        base = s * ROWS_PER_TILE
        is_fast = c == FAST_C
        nb = jnp.where(is_fast, B_FAST, B_SLOW)
        bstart = jnp.where(is_fast, s * B_FAST, 16 * B_FAST + s * B_SLOW)

        @pl.loop(0, RB)
        def _(i):
            for k in range(width // 16):
                zb[i, pl.ds(k * 16, 16)] = jnp.zeros((16,), jnp.float32)
            if with_deg:
                dzb[i, pl.ds(0, 16)] = jnp.zeros((16,), jnp.float32)
                ones[i, pl.ds(0, 16)] = jnp.ones((16,), jnp.float32)

        @pl.loop(0, ROWS_PER_TILE, step=RB)
        def _(r):
            pltpu.sync_copy(zb, acc.at[pl.ds(base + r, RB)])
            if with_deg:
                pltpu.sync_copy(dzb, dacc.at[pl.ds(base + r, RB)])

        pltpu.sync_copy(src_hbm.at[pl.ds(bstart, B_FAST)], srcv)
        pltpu.sync_copy(dst_hbm.at[pl.ds(bstart, B_FAST)], dstv)
        plsc.subcore_barrier()

        @pl.loop(0, nb, step=NBUF)
        def _(j0):
            gds = []
            for b in range(NBUF):
                @pl.when(j0 > 0)
                def _(b=b):
                    pltpu.make_async_copy(
                        rows.at[b], acc.at[dstv.at[j0]], s_sem.at[b]).wait()
                gds.append(pltpu.async_copy(
                    table_hbm.at[srcv.at[j0 + b]], rows.at[b], g_sem.at[b]))
            for b in range(NBUF):
                gds[b].wait()
                pltpu.async_copy(rows.at[b], acc.at[dstv.at[j0 + b]],
                                 s_sem.at[b], add=True)
                if with_deg:
                    od = pltpu.async_copy(ones, dacc.at[dstv.at[j0 + b]],
                                          o_sem, add=True)

                    @pl.when(j0 > 0)
                    def _(od=od):
                        od.wait()

        for b in range(NBUF):
            pltpu.make_async_copy(rows.at[b], acc.at[dstv.at[0]],
                                  s_sem.at[b]).wait()
            if with_deg:
                pltpu.make_async_copy(ones, dacc.at[dstv.at[0]], o_sem).wait()

        plsc.subcore_barrier()

        pltpu.sync_copy(acc.at[pl.ds(base, ROWS_PER_TILE)],
                        agg_hbm.at[c, pl.ds(base, ROWS_PER_TILE)])
        if with_deg:
            pltpu.sync_copy(dacc.at[pl.ds(base, ROWS_PER_TILE)],
                            deg_hbm.at[c, pl.ds(base, ROWS_PER_TILE)])

    kern = pl.kernel(body, out_type=out_types, mesh=_mesh,
                     scratch_types=scratch,
                     compiler_params=pltpu.CompilerParams(
                         use_tc_tiling_on_sc=False))
    return kern(table, src3, dst3)


def _proj1_body(x_ref, w_ref, p_ref, r_ref):
    y = jnp.dot(x_ref[...], w_ref[...], preferred_element_type=jnp.float32)
    p_ref[...] = y[:, :HID]
    r_ref[...] = y[:, HID:]


def _combine1_body(a_ref, d_ref, r1_ref, w2_ref, b1_ref,
                   p2_ref, r2_ref, inv_ref):
    agg = a_ref[0] + a_ref[1]
    deg = d_ref[0] + d_ref[1]
    inv = 1.0 / jnp.maximum(deg, 1.0)
    h = jnp.maximum(agg * inv[:, 0:1] + r1_ref[...] + b1_ref[...], 0.0)
    pr = jnp.dot(h, w2_ref[...], preferred_element_type=jnp.float32)
    p2_ref[...] = pr[:, :OUT_D]
    r2_ref[...] = pr[:, OUT_D:]
    inv_ref[...] = inv


def _combine2_body(a_ref, inv_ref, r2_ref, b2_ref, o_ref):
    agg = a_ref[0] + a_ref[1]
    o_ref[...] = agg * inv_ref[:, 0:1] + r2_ref[...] + b2_ref[...]


def kernel(x, edge_index, Wl1, bl1, Wr1, Wl2, bl2, Wr2):
    src = edge_index[0].astype(jnp.int32)
    dst = edge_index[1].astype(jnp.int32)
    pad = NBT_PAD * BATCH - E
    src3 = jnp.concatenate([src, jnp.zeros((pad,), jnp.int32)]).reshape(
        NBT_PAD, BATCH)
    dst3 = jnp.concatenate([dst, jnp.full((pad,), DUMMY, jnp.int32)]).reshape(
        NBT_PAD, BATCH)

    w1 = jnp.concatenate([Wl1, Wr1], axis=1)
    w2 = jnp.concatenate([Wl2, Wr2], axis=1)
    b1 = bl1.reshape(1, HID)
    b2 = bl2.reshape(1, OUT_D)

    p1, r1 = pl.pallas_call(
        _proj1_body,
        grid=(GRID,),
        in_specs=[
            pl.BlockSpec((BM, IN_D), lambda i: (i, 0)),
            pl.BlockSpec((IN_D, IN_D), lambda i: (0, 0)),
        ],
        out_specs=[
            pl.BlockSpec((BM, HID), lambda i: (i, 0)),
            pl.BlockSpec((BM, HID), lambda i: (i, 0)),
        ],
        out_shape=[
            jax.ShapeDtypeStruct((N, HID), jnp.float32),
            jax.ShapeDtypeStruct((N, HID), jnp.float32),
        ],
    )(x, w1)

    agg1p, degp = _edge_pass(p1, src3, dst3, HID, True)

    p2, r2, invd = pl.pallas_call(
        _combine1_body,
        grid=(GRID,),
        in_specs=[
            pl.BlockSpec((NC, BM, HID), lambda i: (0, i, 0)),
            pl.BlockSpec((NC, BM, 16), lambda i: (0, i, 0)),
            pl.BlockSpec((BM, HID), lambda i: (i, 0)),
            pl.BlockSpec((HID, HID), lambda i: (0, 0)),
            pl.BlockSpec((1, HID), lambda i: (0, 0)),
        ],
        out_specs=[
            pl.BlockSpec((BM, OUT_D), lambda i: (i, 0)),
            pl.BlockSpec((BM, OUT_D), lambda i: (i, 0)),
            pl.BlockSpec((BM, 16), lambda i: (i, 0)),
        ],
        out_shape=[
            jax.ShapeDtypeStruct((N, OUT_D), jnp.float32),
            jax.ShapeDtypeStruct((N, OUT_D), jnp.float32),
            jax.ShapeDtypeStruct((N, 16), jnp.float32),
        ],
    )(agg1p, degp, r1, w2, b1)

    (agg2p,) = _edge_pass(p2, src3, dst3, OUT_D, False)

    out = pl.pallas_call(
        _combine2_body,
        grid=(GRID,),
        in_specs=[
            pl.BlockSpec((NC, BM, OUT_D), lambda i: (0, i, 0)),
            pl.BlockSpec((BM, 16), lambda i: (i, 0)),
            pl.BlockSpec((BM, OUT_D), lambda i: (i, 0)),
            pl.BlockSpec((1, OUT_D), lambda i: (0, 0)),
        ],
        out_specs=pl.BlockSpec((BM, OUT_D), lambda i: (i, 0)),
        out_shape=jax.ShapeDtypeStruct((N, OUT_D), jnp.float32),
    )(agg2p, invd, r2, b2)

    return out


# speedup vs baseline: 7.7340x; 1.0012x over previous
import functools

import jax
import jax.numpy as jnp
from jax import lax
from jax.experimental import pallas as pl
from jax.experimental.pallas import tpu as pltpu
from jax.experimental.pallas import tpu_sc as plsc

N = 10000
E = 320000
IN_D = 128
HID = 64
OUT_D = 32

NC = 2
NS = 16
BATCH = 128
B_FAST = 124
B_SLOW = 36
FAST_C = 0
NBT = 16 * B_FAST + 16 * B_SLOW
NBT_PAD = 16 * B_FAST + 15 * B_SLOW + B_FAST
N_PAD = 10240
ROWS_PER_TILE = N_PAD // NS
RB = 128
NBUF = 4
DUMMY = N

BM = 1000
GRID = N // BM

_mesh = plsc.VectorSubcoreMesh(core_axis_name="c", subcore_axis_name="s")


def _edge_pass(table, src3, dst3, width, with_deg):
    out_types = [jax.ShapeDtypeStruct((NC, N_PAD, width), jnp.float32)]
    scratch = [
        pltpu.VMEM((B_FAST, BATCH), jnp.int32),
        pltpu.VMEM((B_FAST, BATCH), jnp.int32),
        pltpu.VMEM((NBUF, BATCH, width), jnp.float32),
        pltpu.VMEM((RB, width), jnp.float32),
        pltpu.VMEM_SHARED((N_PAD, width), jnp.float32),
        pltpu.SemaphoreType.DMA((NBUF,)),
        pltpu.SemaphoreType.DMA((NBUF,)),
        pltpu.SemaphoreType.DMA,
    ]
    if with_deg:
        out_types.append(jax.ShapeDtypeStruct((NC, N_PAD, 16), jnp.float32))
        scratch += [
            pltpu.VMEM((BATCH, 16), jnp.float32),
            pltpu.VMEM((RB, 16), jnp.float32),
            pltpu.VMEM_SHARED((N_PAD, 16), jnp.float32),
        ]

    def body(table_hbm, src_hbm, dst_hbm, *refs):
        if with_deg:
            (agg_hbm, deg_hbm, srcv, dstv, rows, zb, acc,
             g_sem, s_sem, o_sem, ones, dzb, dacc) = refs
        else:
            (agg_hbm, srcv, dstv, rows, zb, acc, g_sem, s_sem, o_sem) = refs
        c = lax.axis_index("c")
        s = lax.axis_index("s")
        base = s * ROWS_PER_TILE
        is_fast = c == FAST_C
        nb = jnp.where(is_fast, B_FAST, B_SLOW)
        bstart = jnp.where(is_fast, s * B_FAST, 16 * B_FAST + s * B_SLOW)

        with jax.named_scope("sc_init"):
            @pl.loop(0, RB)
            def _(i):
                for k in range(width // 16):
                    zb[i, pl.ds(k * 16, 16)] = jnp.zeros((16,), jnp.float32)
                if with_deg:
                    dzb[i, pl.ds(0, 16)] = jnp.zeros((16,), jnp.float32)
                    ones[i, pl.ds(0, 16)] = jnp.ones((16,), jnp.float32)

            @pl.loop(0, ROWS_PER_TILE, step=RB)
            def _(r):
                pltpu.sync_copy(zb, acc.at[pl.ds(base + r, RB)])
                if with_deg:
                    pltpu.sync_copy(dzb, dacc.at[pl.ds(base + r, RB)])

            pltpu.sync_copy(src_hbm.at[pl.ds(bstart, B_FAST)], srcv)
            pltpu.sync_copy(dst_hbm.at[pl.ds(bstart, B_FAST)], dstv)
            plsc.subcore_barrier()

        _edges_scope = jax.named_scope("sc_edges")
        _edges_scope.__enter__()

        @pl.loop(0, nb, step=NBUF)
        def _(j0):
            gds = []
            for b in range(NBUF):
                @pl.when(j0 > 0)
                def _(b=b):
                    pltpu.make_async_copy(
                        rows.at[b], acc.at[dstv.at[j0]], s_sem.at[b]).wait()
                gds.append(pltpu.async_copy(
                    table_hbm.at[srcv.at[j0 + b]], rows.at[b], g_sem.at[b]))
            for b in range(NBUF):
                gds[b].wait()
                pltpu.async_copy(rows.at[b], acc.at[dstv.at[j0 + b]],
                                 s_sem.at[b], add=True)
                if with_deg:
                    od = pltpu.async_copy(ones, dacc.at[dstv.at[j0 + b]],
                                          o_sem, add=True)

                    @pl.when(j0 > 0)
                    def _(od=od):
                        od.wait()

        for b in range(NBUF):
            pltpu.make_async_copy(rows.at[b], acc.at[dstv.at[0]],
                                  s_sem.at[b]).wait()
            if with_deg:
                pltpu.make_async_copy(ones, dacc.at[dstv.at[0]], o_sem).wait()

        plsc.subcore_barrier()
        _edges_scope.__exit__(None, None, None)

        with jax.named_scope("sc_wb"):
            pltpu.sync_copy(acc.at[pl.ds(base, ROWS_PER_TILE)],
                            agg_hbm.at[c, pl.ds(base, ROWS_PER_TILE)])
            if with_deg:
                pltpu.sync_copy(dacc.at[pl.ds(base, ROWS_PER_TILE)],
                                deg_hbm.at[c, pl.ds(base, ROWS_PER_TILE)])

    kern = pl.kernel(body, out_type=out_types, mesh=_mesh,
                     scratch_types=scratch,
                     compiler_params=pltpu.CompilerParams(
                         use_tc_tiling_on_sc=False))
    return kern(table, src3, dst3)


def _proj1_body(x_ref, w_ref, p_ref, r_ref):
    y = jnp.dot(x_ref[...], w_ref[...], preferred_element_type=jnp.float32)
    p_ref[...] = y[:, :HID]
    r_ref[...] = y[:, HID:]


def _combine1_body(a_ref, d_ref, r1_ref, w2_ref, b1_ref,
                   p2_ref, r2_ref, inv_ref):
    agg = a_ref[0] + a_ref[1]
    deg = d_ref[0] + d_ref[1]
    inv = 1.0 / jnp.maximum(deg, 1.0)
    h = jnp.maximum(agg * inv[:, 0:1] + r1_ref[...] + b1_ref[...], 0.0)
    pr = jnp.dot(h, w2_ref[...], preferred_element_type=jnp.float32)
    p2_ref[...] = pr[:, :OUT_D]
    r2_ref[...] = pr[:, OUT_D:]
    inv_ref[...] = inv


def _combine2_body(a_ref, inv_ref, r2_ref, b2_ref, o_ref):
    agg = a_ref[0] + a_ref[1]
    o_ref[...] = agg * inv_ref[:, 0:1] + r2_ref[...] + b2_ref[...]


def kernel(x, edge_index, Wl1, bl1, Wr1, Wl2, bl2, Wr2):
    src = edge_index[0].astype(jnp.int32)
    dst = edge_index[1].astype(jnp.int32)
    pad = NBT_PAD * BATCH - E
    src3 = jnp.concatenate([src, jnp.zeros((pad,), jnp.int32)]).reshape(
        NBT_PAD, BATCH)
    dst3 = jnp.concatenate([dst, jnp.full((pad,), DUMMY, jnp.int32)]).reshape(
        NBT_PAD, BATCH)

    w1 = jnp.concatenate([Wl1, Wr1], axis=1)
    w2 = jnp.concatenate([Wl2, Wr2], axis=1)
    b1 = bl1.reshape(1, HID)
    b2 = bl2.reshape(1, OUT_D)

    p1, r1 = pl.pallas_call(
        _proj1_body,
        grid=(GRID,),
        in_specs=[
            pl.BlockSpec((BM, IN_D), lambda i: (i, 0)),
            pl.BlockSpec((IN_D, IN_D), lambda i: (0, 0)),
        ],
        out_specs=[
            pl.BlockSpec((BM, HID), lambda i: (i, 0)),
            pl.BlockSpec((BM, HID), lambda i: (i, 0)),
        ],
        out_shape=[
            jax.ShapeDtypeStruct((N, HID), jnp.float32),
            jax.ShapeDtypeStruct((N, HID), jnp.float32),
        ],
    )(x, w1)

    agg1p, degp = _edge_pass(p1, src3, dst3, HID, True)

    p2, r2, invd = pl.pallas_call(
        _combine1_body,
        grid=(GRID,),
        in_specs=[
            pl.BlockSpec((NC, BM, HID), lambda i: (0, i, 0)),
            pl.BlockSpec((NC, BM, 16), lambda i: (0, i, 0)),
            pl.BlockSpec((BM, HID), lambda i: (i, 0)),
            pl.BlockSpec((HID, HID), lambda i: (0, 0)),
            pl.BlockSpec((1, HID), lambda i: (0, 0)),
        ],
        out_specs=[
            pl.BlockSpec((BM, OUT_D), lambda i: (i, 0)),
            pl.BlockSpec((BM, OUT_D), lambda i: (i, 0)),
            pl.BlockSpec((BM, 16), lambda i: (i, 0)),
        ],
        out_shape=[
            jax.ShapeDtypeStruct((N, OUT_D), jnp.float32),
            jax.ShapeDtypeStruct((N, OUT_D), jnp.float32),
            jax.ShapeDtypeStruct((N, 16), jnp.float32),
        ],
    )(agg1p, degp, r1, w2, b1)

    (agg2p,) = _edge_pass(p2, src3, dst3, OUT_D, False)

    out = pl.pallas_call(
        _combine2_body,
        grid=(GRID,),
        in_specs=[
            pl.BlockSpec((NC, BM, OUT_D), lambda i: (0, i, 0)),
            pl.BlockSpec((BM, 16), lambda i: (i, 0)),
            pl.BlockSpec((BM, OUT_D), lambda i: (i, 0)),
            pl.BlockSpec((1, OUT_D), lambda i: (0, 0)),
        ],
        out_specs=pl.BlockSpec((BM, OUT_D), lambda i: (i, 0)),
        out_shape=jax.ShapeDtypeStruct((N, OUT_D), jnp.float32),
    )(agg2p, invd, r2, b2)

    return out
